# baseline (speedup 1.0000x reference)
import functools

import jax
import jax.numpy as jnp
from jax import lax
from jax.experimental import pallas as pl
from jax.experimental.pallas import tpu as pltpu
from jax.experimental.pallas import tpu_sc as plsc

N = 100000
G = 1000
E = 3200000
RB = 2000
NBLK = N // RB

_f32 = jnp.float32


def _row_specs(*dims):
    return [pl.BlockSpec((RB, d), lambda i: (i, 0)) for d in dims]


def _split_spec(d=16):
    return pl.BlockSpec((2, RB, d), lambda i: (0, i, 0))


def _full_spec(shape):
    nd = len(shape)
    return pl.BlockSpec(shape, lambda i: (0,) * nd)


def _dot(a, b):
    return jax.lax.dot_general(a, b, (((1,), (0,)), ((), ())),
                               preferred_element_type=_f32,
                               precision=jax.lax.Precision.DEFAULT)


def _prep_body(dp_ref, x_ref, dinv_ref, xs0_ref):
    deg = dp_ref[0] + dp_ref[1] + 1.0
    dinv = lax.rsqrt(deg)
    dinv_ref[...] = dinv
    xs = x_ref[...] * dinv
    xs0_ref[...] = jnp.concatenate(
        [xs, jnp.zeros((RB, 11), _f32)], axis=1)


def _prep(deg_parts, x):
    return pl.pallas_call(
        _prep_body,
        grid=(NBLK,),
        in_specs=[_split_spec(1)] + _row_specs(5),
        out_specs=_row_specs(1, 16),
        out_shape=[jax.ShapeDtypeStruct((N, 1), _f32),
                   jax.ShapeDtypeStruct((N, 16), _f32)],
    )(deg_parts, x)


def _layer16_body(acc_ref, xs_ref, dinv_ref, W_ref, b_ref, out_ref):
    dinv = dinv_ref[...]
    t = dinv * (acc_ref[0] + acc_ref[1] + xs_ref[...])
    h = jnp.maximum(_dot(t, W_ref[...]) + b_ref[...], 0.0)
    xso = dinv * h
    out_ref[0] = xso[:, :16]
    out_ref[1] = xso[:, 16:]


def _layer16(acc_parts, xs, dinv, Wp, b):
    return pl.pallas_call(
        _layer16_body,
        grid=(NBLK,),
        in_specs=[_split_spec(), *_row_specs(16, 1),
                  _full_spec((16, 32)), _full_spec((1, 32))],
        out_specs=_split_spec(),
        out_shape=jax.ShapeDtypeStruct((2, N, 16), _f32),
    )(acc_parts, xs, dinv, Wp, b)


def _layer32_body(acc_ref, xs_ref, dinv_ref, W_ref, b_ref, out_ref):
    dinv = dinv_ref[...]
    s = jnp.concatenate([acc_ref[0], acc_ref[1]], axis=1)
    xsc = jnp.concatenate([xs_ref[0], xs_ref[1]], axis=1)
    t = dinv * (s + xsc)
    h = jnp.maximum(_dot(t, W_ref[...]) + b_ref[...], 0.0)
    xso = dinv * h
    out_ref[0] = xso[:, :16]
    out_ref[1] = xso[:, 16:]


def _layer32(acc, xs, dinv, W, b):
    return pl.pallas_call(
        _layer32_body,
        grid=(NBLK,),
        in_specs=[_split_spec(), _split_spec(), *_row_specs(1),
                  _full_spec((32, 32)), _full_spec((1, 32))],
        out_specs=_split_spec(),
        out_shape=jax.ShapeDtypeStruct((2, N, 16), _f32),
    )(acc, xs, dinv, W, b)


def _enc4_body(acc_ref, xs_ref, dinv_ref, W_ref, b_ref,
               mW1_ref, mb1_ref, mW2_ref, mb2_ref,
               gW1_ref, gb1_ref, gW2_ref, gb2_ref, out_ref):
    dinv = dinv_ref[...]
    s = jnp.concatenate([acc_ref[0], acc_ref[1]], axis=1)
    xsc = jnp.concatenate([xs_ref[0], xs_ref[1]], axis=1)
    t = dinv * (s + xsc)
    h = jnp.maximum(_dot(t, W_ref[...]) + b_ref[...], 0.0)
    m = _dot(jnp.maximum(_dot(h, mW1_ref[...]) + mb1_ref[...], 0.0),
             mW2_ref[...]) + mb2_ref[...]
    g = _dot(jnp.maximum(_dot(h, gW1_ref[...]) + gb1_ref[...], 0.0),
             gW2_ref[...]) + gb2_ref[...]
    out_ref[...] = jnp.concatenate(
        [m, g, jnp.ones((RB, 1), _f32), jnp.zeros((RB, 9), _f32)], axis=1)


def _enc4_head(acc, xs, dinv, W, b, mW1, mb1, mW2, mb2, gW1, gb1, gW2, gb2):
    return pl.pallas_call(
        _enc4_body,
        grid=(NBLK,),
        in_specs=[_split_spec(), _split_spec(), *_row_specs(1),
                  _full_spec((32, 32)), _full_spec((1, 32)),
                  _full_spec((32, 16)), _full_spec((1, 16)),
                  _full_spec((16, 3)), _full_spec((1, 3)),
                  _full_spec((32, 16)), _full_spec((1, 16)),
                  _full_spec((16, 3)), _full_spec((1, 3))],
        out_specs=_row_specs(16)[0],
        out_shape=jax.ShapeDtypeStruct((N, 16), _f32),
    )(acc, xs, dinv, W, b, mW1, mb1, mW2, mb2, gW1, gb1, gW2, gb2)


def _z_body(pool_ref, eps_ref, mu_ref, sg_ref, zp_ref):
    p = pool_ref[0] + pool_ref[1]
    denom = jnp.maximum(p[:, 6:7], 1.0)
    mu = p[:, 0:3] / denom
    sg = p[:, 3:6] / denom
    z = mu + eps_ref[...] * jnp.exp(0.5 * sg)
    mu_ref[...] = mu
    sg_ref[...] = sg
    zp_ref[...] = jnp.concatenate([z, jnp.zeros((G, 13), _f32)], axis=1)


def _z_kernel(pooled, eps):
    return pl.pallas_call(
        _z_body,
        in_specs=[pl.BlockSpec((2, G, 16), lambda: (0, 0, 0)),
                  pl.BlockSpec((G, 3), lambda: (0, 0))],
        out_specs=[pl.BlockSpec((G, 3), lambda: (0, 0)),
                   pl.BlockSpec((G, 3), lambda: (0, 0)),
                   pl.BlockSpec((G, 16), lambda: (0, 0))],
        out_shape=[jax.ShapeDtypeStruct((G, 3), _f32),
                   jax.ShapeDtypeStruct((G, 3), _f32),
                   jax.ShapeDtypeStruct((G, 16), _f32)],
    )(pooled, eps)


def _dec_head_body(zn_ref, dinv_ref, W1_ref, b1_ref, W2_ref, b2_ref, out_ref):
    h = jnp.maximum(_dot(zn_ref[...], W1_ref[...]) + b1_ref[...], 0.0)
    h2 = jnp.maximum(_dot(h, W2_ref[...]) + b2_ref[...], 0.0)
    xso = dinv_ref[...] * h2
    out_ref[0] = xso[:, :16]
    out_ref[1] = xso[:, 16:]


def _dec_head(zn, dinv, W1p, b1, W2, b2):
    return pl.pallas_call(
        _dec_head_body,
        grid=(NBLK,),
        in_specs=[*_row_specs(16, 1), _full_spec((16, 16)),
                  _full_spec((1, 16)), _full_spec((16, 32)),
                  _full_spec((1, 32))],
        out_specs=_split_spec(),
        out_shape=jax.ShapeDtypeStruct((2, N, 16), _f32),
    )(zn, dinv, W1p, b1, W2, b2)


def _dec3_body(acc_ref, xs_ref, dinv_ref, W_ref, b_ref, W4_ref, out_ref):
    dinv = dinv_ref[...]
    s = jnp.concatenate([acc_ref[0], acc_ref[1]], axis=1)
    xsc = jnp.concatenate([xs_ref[0], xs_ref[1]], axis=1)
    t = dinv * (s + xsc)
    h = jnp.maximum(_dot(t, W_ref[...]) + b_ref[...], 0.0)
    out_ref[...] = _dot(dinv * h, W4_ref[...])


def _dec3(acc, xs, dinv, W3, b3, W4p):
    return pl.pallas_call(
        _dec3_body,
        grid=(NBLK,),
        in_specs=[_split_spec(), _split_spec(), *_row_specs(1),
                  _full_spec((32, 32)), _full_spec((1, 32)),
                  _full_spec((32, 16))],
        out_specs=_row_specs(16)[0],
        out_shape=jax.ShapeDtypeStruct((N, 16), _f32),
    )(acc, xs, dinv, W3, b3, W4p)


def _final_body(acc_ref, ys_ref, dinv_ref, b_ref, out_ref):
    t = dinv_ref[...] * (acc_ref[0] + acc_ref[1] + ys_ref[...]) + b_ref[...]
    out_ref[...] = jnp.maximum(t, 0.0)[:, :5]


def _final(acc_parts, ys, dinv, b4p):
    return pl.pallas_call(
        _final_body,
        grid=(NBLK,),
        in_specs=[_split_spec(), *_row_specs(16, 1), _full_spec((1, 16))],
        out_specs=_row_specs(5)[0],
        out_shape=jax.ShapeDtypeStruct((N, 5), _f32),
    )(acc_parts, ys, dinv, b4p)


E_PAD = 3211264
IROWS = E_PAD // 128
ACC_R = 100352
DUMMY = 100000
CWIN = 4

_mesh = plsc.VectorSubcoreMesh(core_axis_name="c", subcore_axis_name="s")


def _edge_pass(xs, si_pad, di_pad, col_split):
    n_chunks = (IROWS // 16 if col_split else IROWS // 32) // CWIN

    @functools.partial(
        pl.kernel,
        compiler_params=pltpu.CompilerParams(use_tc_tiling_on_sc=False),
        out_type=jax.ShapeDtypeStruct((2, N, 16), _f32),
        mesh=_mesh,
        scratch_types=[
            pltpu.VMEM((2, CWIN * 128), jnp.int32),
            pltpu.VMEM((2, CWIN * 128), jnp.int32),
            pltpu.VMEM((2, CWIN * 128, 16), _f32),
            pltpu.VMEM((128, 16), _f32),
            pltpu.VMEM_SHARED((ACC_R, 16), _f32),
            pltpu.SemaphoreType.DMA,
            pltpu.SemaphoreType.DMA,
            pltpu.SemaphoreType.DMA,
            pltpu.SemaphoreType.DMA,
            pltpu.SemaphoreType.DMA,
        ],
    )
    def k(xs_hbm, si_hbm, di_hbm, out_hbm,
          si_v, di_v, rows_v, zb_v, acc,
          sem_i0, sem_i1, sem_g, sem_s0, sem_s1):
        c = lax.axis_index("c")
        s = lax.axis_index("s")

        @pl.loop(0, 128)
        def _fill(i):
            zb_v[i, :] = jnp.zeros((16,), _f32)

        @pl.loop(0, 49)
        def _zero(i):
            pltpu.sync_copy(zb_v, acc.at[pl.ds(s * 6272 + i * 128, 128)])

        plsc.subcore_barrier()

        ew = CWIN * 128
        base_e = (s * 1568 if col_split else (s * 2 + c) * 784) * 128
        sem_i = (sem_i0, sem_i1)
        sem_s = (sem_s0, sem_s1)
        src_tbl = xs_hbm.at[c] if col_split else xs_hbm
        last = n_chunks - 1

        def stage_idx(i, b):
            e0 = base_e + jnp.minimum(i, last) * ew
            pltpu.async_copy(si_hbm.at[pl.ds(e0, ew)], si_v.at[b],
                             sem_i[b])
            pltpu.async_copy(di_hbm.at[pl.ds(e0, ew)], di_v.at[b],
                             sem_i[b])

        def wait_idx(i, b):
            e0 = base_e + jnp.minimum(i, last) * ew
            pltpu.make_async_copy(si_hbm.at[pl.ds(e0, ew)], si_v.at[b],
                                  sem_i[b]).wait()
            pltpu.make_async_copy(di_hbm.at[pl.ds(e0, ew)], di_v.at[b],
                                  sem_i[b]).wait()

        def wait_scat(b):
            pltpu.make_async_copy(rows_v.at[b], acc.at[di_v.at[b]],
                                  sem_s[b]).wait()

        def do_chunk(i, b, first):
            if not first:
                wait_scat(b)
            wait_idx(i, b)
            pltpu.async_copy(src_tbl.at[si_v.at[b]], rows_v.at[b],
                             sem_g).wait()
            pltpu.async_copy(rows_v.at[b], acc.at[di_v.at[b]], sem_s[b],
                             add=True)
            stage_idx(i + 2, b)

        stage_idx(0, 0)
        stage_idx(1, 1)
        do_chunk(0, 0, True)
        do_chunk(1, 1, True)

        @pl.loop(2, n_chunks)
        def _c(i):
            @pl.when(i % 2 == 0)
            def _even():
                do_chunk(i, 0, False)

            @pl.when(i % 2 == 1)
            def _odd():
                do_chunk(i, 1, False)

        wait_scat(0)
        wait_scat(1)
        wait_idx(n_chunks, 0)
        wait_idx(n_chunks + 1, 1)
        plsc.subcore_barrier()

        @pl.when(s < 15)
        def _drain():
            pltpu.sync_copy(acc.at[pl.ds(s * 6256, 6256)],
                            out_hbm.at[c].at[pl.ds(s * 6256, 6256)])

        @pl.when(s == 15)
        def _drain_tail():
            pltpu.sync_copy(acc.at[pl.ds(93840, 6160)],
                            out_hbm.at[c].at[pl.ds(93840, 6160)])

    return k(xs, si_pad, di_pad)


def _sc_deg(di_pad):
    n_chunks = (IROWS // 32) // CWIN

    @functools.partial(
        pl.kernel,
        compiler_params=pltpu.CompilerParams(use_tc_tiling_on_sc=False),
        out_type=[jax.ShapeDtypeStruct((ACC_R,), _f32),
                  jax.ShapeDtypeStruct((ACC_R,), _f32)],
        mesh=_mesh,
        scratch_types=[
            pltpu.VMEM((CWIN, 128), jnp.int32),
            pltpu.VMEM((128,), _f32),
            pltpu.VMEM((784,), _f32),
            pltpu.VMEM_SHARED((ACC_R,), _f32),
            pltpu.SemaphoreType.DMA,
            pltpu.SemaphoreType.DMA,
        ],
    )
    def k(di_hbm, out0_hbm, out1_hbm, di_v, ones_v, zb_v, acc, sem_i, sem_s):
        c = lax.axis_index("c")
        s = lax.axis_index("s")

        @pl.loop(0, 8)
        def _fill1(i):
            ones_v[pl.ds(i * 16, 16)] = jnp.ones((16,), _f32)

        @pl.loop(0, 49)
        def _fill0(i):
            zb_v[pl.ds(i * 16, 16)] = jnp.zeros((16,), _f32)

        @pl.loop(0, 8)
        def _zero(i):
            pltpu.sync_copy(zb_v, acc.at[pl.ds(s * 6272 + i * 784, 784)])

        plsc.subcore_barrier()
        base_row = (s * 2 + c) * 784

        @pl.loop(0, n_chunks)
        def _chunk(i):
            row0 = base_row + i * CWIN
            pltpu.async_copy(di_hbm.at[pl.ds(row0, CWIN)], di_v, sem_i).wait()
            ss = [pltpu.async_copy(ones_v, acc.at[di_v.at[j]], sem_s,
                                   add=True) for j in range(CWIN)]
            for t in ss:
                t.wait()

        plsc.subcore_barrier()

        @pl.when(c == 0)
        def _drain0():
            pltpu.sync_copy(acc.at[pl.ds(s * 6272, 6272)],
                            out0_hbm.at[pl.ds(s * 6272, 6272)])

        @pl.when(c == 1)
        def _drain1():
            pltpu.sync_copy(acc.at[pl.ds(s * 6272, 6272)],
                            out1_hbm.at[pl.ds(s * 6272, 6272)])

    return k(di_pad)


NP = 131072
GDUM = 1000


def _sc_pool(combined_pad, bi_rows):
    @functools.partial(
        pl.kernel,
        compiler_params=pltpu.CompilerParams(use_tc_tiling_on_sc=False),
        out_type=jax.ShapeDtypeStruct((2, G, 16), _f32),
        mesh=_mesh,
        scratch_types=[
            pltpu.VMEM((32, 128), jnp.int32),
            pltpu.VMEM((4096, 16), _f32),
            pltpu.VMEM((64, 16), _f32),
            pltpu.VMEM_SHARED((1024, 16), _f32),
            pltpu.SemaphoreType.DMA,
            pltpu.SemaphoreType.DMA,
        ],
    )
    def k(comb_hbm, bi_hbm, out_hbm, bi_v, rows_v, zb_v, acc, sem_i, sem_s):
        c = lax.axis_index("c")
        s = lax.axis_index("s")

        @pl.loop(0, 64)
        def _fill(i):
            zb_v[i, :] = jnp.zeros((16,), _f32)

        pltpu.sync_copy(zb_v, acc.at[pl.ds(s * 64, 64)])
        plsc.subcore_barrier()

        w = s * 2 + c
        ci = pltpu.async_copy(bi_hbm.at[pl.ds(w * 32, 32)], bi_v, sem_i)
        cr = pltpu.async_copy(comb_hbm.at[pl.ds(w * 4096, 4096)], rows_v,
                              sem_i)
        ci.wait()
        cr.wait()
        ss = [pltpu.async_copy(rows_v.at[pl.ds(j * 128, 128)],
                               acc.at[bi_v.at[j]], sem_s, add=True)
              for j in range(32)]
        for t in ss:
            t.wait()
        plsc.subcore_barrier()

        @pl.when(s == 0)
        def _drain():
            pltpu.sync_copy(acc.at[pl.ds(0, G)], out_hbm.at[c])

    return k(combined_pad, bi_rows)


def _sc_zn(zpad, bi_rows):
    @functools.partial(
        pl.kernel,
        compiler_params=pltpu.CompilerParams(use_tc_tiling_on_sc=False),
        out_type=jax.ShapeDtypeStruct((NP, 16), _f32),
        mesh=_mesh,
        scratch_types=[
            pltpu.VMEM((32, 128), jnp.int32),
            pltpu.VMEM((4096, 16), _f32),
            pltpu.SemaphoreType.DMA,
            pltpu.SemaphoreType.DMA,
        ],
    )
    def k(z_hbm, bi_hbm, out_hbm, bi_v, rows_v, sem_i, sem_g):
        c = lax.axis_index("c")
        s = lax.axis_index("s")
        w = s * 2 + c
        pltpu.async_copy(bi_hbm.at[pl.ds(w * 32, 32)], bi_v, sem_i).wait()
        gs = [pltpu.async_copy(z_hbm.at[bi_v.at[j]],
                               rows_v.at[pl.ds(j * 128, 128)], sem_g)
              for j in range(32)]
        for g in gs:
            g.wait()
        pltpu.sync_copy(rows_v, out_hbm.at[pl.ds(w * 4096, 4096)])

    return k(zpad, bi_rows)


def kernel(x, edge_index, batch_index,
           enc_W1, enc_b1, enc_W2, enc_b2, enc_W3, enc_b3, enc_W4, enc_b4,
           mu_W1, mu_b1, mu_W2, mu_b2,
           sg_W1, sg_b1, sg_W2, sg_b2,
           un_W1, un_b1, un_W2, un_b2,
           dec_W1, dec_b1, dec_W2, dec_b2, dec_W3, dec_b3, dec_W4, dec_b4):
    src = edge_index[0].astype(jnp.int32)
    dst = edge_index[1].astype(jnp.int32)

    W1p = jnp.zeros((16, 32), _f32).at[:5].set(enc_W1)
    unW1p = jnp.zeros((16, 16), _f32).at[:3].set(un_W1)
    decW4p = jnp.zeros((32, 16), _f32).at[:, :5].set(dec_W4)
    b4p = jnp.zeros((1, 16), _f32).at[0, :5].set(dec_b4)
    eps = jax.random.normal(jax.random.key(42), (G, 3), dtype=_f32)

    npad = E_PAD - E
    si_pad = jnp.concatenate([src, jnp.zeros((npad,), jnp.int32)])
    di_pad = jnp.concatenate([dst, jnp.full((npad,), DUMMY, jnp.int32)])
    bi_pad = jnp.concatenate(
        [batch_index.astype(jnp.int32),
         jnp.full((NP - N,), GDUM, jnp.int32)]).reshape(1024, 128)

    deg0, deg1 = _sc_deg(di_pad.reshape(IROWS, 128))
    deg_parts = jnp.stack([deg0[:N], deg1[:N]]).reshape(2, N, 1)
    dinv, xs0 = _prep(deg_parts, x)

    acc = _edge_pass(xs0, si_pad, di_pad, col_split=False)
    xs = _layer16(acc, xs0, dinv, W1p, enc_b1.reshape(1, 32))
    for W, b in ((enc_W2, enc_b2), (enc_W3, enc_b3)):
        acc = _edge_pass(xs, si_pad, di_pad, col_split=True)
        xs = _layer32(acc, xs, dinv, W, b.reshape(1, 32))
    acc = _edge_pass(xs, si_pad, di_pad, col_split=True)
    combined = _enc4_head(acc, xs, dinv, enc_W4, enc_b4.reshape(1, 32),
                          mu_W1, mu_b1.reshape(1, 16), mu_W2,
                          mu_b2.reshape(1, 3),
                          sg_W1, sg_b1.reshape(1, 16), sg_W2,
                          sg_b2.reshape(1, 3))

    combined_pad = jnp.concatenate(
        [combined, jnp.zeros((NP - N, 16), _f32)])
    pooled = _sc_pool(combined_pad, bi_pad)
    mu, sigma, zpad = _z_kernel(pooled, eps)

    zpad2 = jnp.concatenate([zpad, jnp.zeros((8, 16), _f32)])
    zn = _sc_zn(zpad2, bi_pad)[:N]
    xs = _dec_head(zn, dinv, unW1p, un_b1.reshape(1, 16), un_W2,
                   un_b2.reshape(1, 32))
    for W, b in ((dec_W1, dec_b1), (dec_W2, dec_b2)):
        acc = _edge_pass(xs, si_pad, di_pad, col_split=True)
        xs = _layer32(acc, xs, dinv, W, b.reshape(1, 32))
    acc = _edge_pass(xs, si_pad, di_pad, col_split=True)
    ys = _dec3(acc, xs, dinv, dec_W3, dec_b3.reshape(1, 32), decW4p)
    acc = _edge_pass(ys, si_pad, di_pad, col_split=False)
    h2 = _final(acc, ys, dinv, b4p)
    return (h2, mu, sigma)


# speedup vs baseline: 1.2883x; 1.2883x over previous
import functools

import jax
import jax.numpy as jnp
from jax import lax
from jax.experimental import pallas as pl
from jax.experimental.pallas import tpu as pltpu
from jax.experimental.pallas import tpu_sc as plsc

N = 100000
G = 1000
E = 3200000

E_PAD = 3211264
IROWS = E_PAD // 128
ACC_R = 100352
NPK = ACC_R
PR = NPK // 8
PB = 256
PGRID = PR // PB
DUMMY = 100000
CWIN = 4
NP = 131072
GDUM = 1000

_f32 = jnp.float32

_mesh = plsc.VectorSubcoreMesh(core_axis_name="c", subcore_axis_name="s")


def _pspec():
    return pl.BlockSpec((PB, 128), lambda i: (i, 0))


def _pspec2():
    return pl.BlockSpec((2, PB, 128), lambda i: (0, i, 0))


def _wspec():
    return pl.BlockSpec((128, 128), lambda i: (0, 0))


def _bspec():
    return pl.BlockSpec((1, 128), lambda i: (0, 0))


def _pout(n=1):
    if n == 1:
        return _pspec(), jax.ShapeDtypeStruct((PR, 128), _f32)
    return _pspec2(), jax.ShapeDtypeStruct((2, PR, 128), _f32)


def _dot(a, b):
    return jax.lax.dot_general(a, b, (((1,), (0,)), ((), ())),
                               preferred_element_type=_f32,
                               precision=jax.lax.Precision.DEFAULT)


def _prep_body(d0_ref, d1_ref, xp_ref, dinv_ref, xs0_ref):
    dinv = lax.rsqrt(d0_ref[...] + d1_ref[...] + 1.0)
    dinv_ref[...] = dinv
    xs0_ref[...] = xp_ref[...] * dinv


def _prep(deg0_p, deg1_p, xpad_p):
    os, osh = _pout()
    return pl.pallas_call(
        _prep_body,
        grid=(PGRID,),
        in_specs=[_pspec(), _pspec(), _pspec()],
        out_specs=[os, os],
        out_shape=[osh, osh],
    )(deg0_p, deg1_p, xpad_p)


def _layer16_body(acc_ref, xs_ref, dinv_ref, Wl_ref, Wh_ref, bl_ref, bh_ref,
                  out_ref):
    dinv = dinv_ref[...]
    t = dinv * (acc_ref[0] + acc_ref[1] + xs_ref[...])
    out_ref[0] = dinv * jnp.maximum(_dot(t, Wl_ref[...]) + bl_ref[...], 0.0)
    out_ref[1] = dinv * jnp.maximum(_dot(t, Wh_ref[...]) + bh_ref[...], 0.0)


def _layer16(acc_p, xs_p, dinv_p, BDl, BDh, bl, bh):
    os, osh = _pout(2)
    return pl.pallas_call(
        _layer16_body,
        grid=(PGRID,),
        in_specs=[_pspec2(), _pspec(), _pspec(),
                  _wspec(), _wspec(), _bspec(), _bspec()],
        out_specs=os,
        out_shape=osh,
    )(acc_p, xs_p, dinv_p, BDl, BDh, bl, bh)


def _hidden32(acc_ref, xs_ref, dinv, Wll, Whl, Wlh, Whh, bl, bh):
    t0 = dinv * (acc_ref[0] + xs_ref[0])
    t1 = dinv * (acc_ref[1] + xs_ref[1])
    h0 = jnp.maximum(_dot(t0, Wll) + _dot(t1, Whl) + bl, 0.0)
    h1 = jnp.maximum(_dot(t0, Wlh) + _dot(t1, Whh) + bh, 0.0)
    return h0, h1


def _layer32_body(acc_ref, xs_ref, dinv_ref, Wll_ref, Whl_ref, Wlh_ref,
                  Whh_ref, bl_ref, bh_ref, out_ref):
    dinv = dinv_ref[...]
    h0, h1 = _hidden32(acc_ref, xs_ref, dinv, Wll_ref[...], Whl_ref[...],
                       Wlh_ref[...], Whh_ref[...], bl_ref[...], bh_ref[...])
    out_ref[0] = dinv * h0
    out_ref[1] = dinv * h1


def _layer32(acc_p, xs_p, dinv_p, BDs, bl, bh):
    os, osh = _pout(2)
    return pl.pallas_call(
        _layer32_body,
        grid=(PGRID,),
        in_specs=[_pspec2(), _pspec2(), _pspec(),
                  _wspec(), _wspec(), _wspec(), _wspec(),
                  _bspec(), _bspec()],
        out_specs=os,
        out_shape=osh,
    )(acc_p, xs_p, dinv_p, *BDs, bl, bh)


def _enc4_body(acc_ref, xs_ref, dinv_ref, Wll_ref, Whl_ref, Wlh_ref, Whh_ref,
               bl_ref, bh_ref, Ml_ref, Mh_ref, bm1_ref, M2_ref, bm2_ref,
               Gl_ref, Gh_ref, bg1_ref, G2_ref, bg2_ref, SH_ref, one_ref,
               out_ref):
    dinv = dinv_ref[...]
    h0, h1 = _hidden32(acc_ref, xs_ref, dinv, Wll_ref[...], Whl_ref[...],
                       Wlh_ref[...], Whh_ref[...], bl_ref[...], bh_ref[...])
    m1 = jnp.maximum(_dot(h0, Ml_ref[...]) + _dot(h1, Mh_ref[...])
                     + bm1_ref[...], 0.0)
    mu = _dot(m1, M2_ref[...]) + bm2_ref[...]
    g1 = jnp.maximum(_dot(h0, Gl_ref[...]) + _dot(h1, Gh_ref[...])
                     + bg1_ref[...], 0.0)
    sg = _dot(g1, G2_ref[...]) + bg2_ref[...]
    out_ref[...] = mu + _dot(sg, SH_ref[...]) + one_ref[...]


def _enc4_head(acc_p, xs_p, dinv_p, BDs, bl, bh, Ml, Mh, bm1, M2, bm2,
               Gl, Gh, bg1, G2, bg2, SH, one6):
    os, osh = _pout()
    return pl.pallas_call(
        _enc4_body,
        grid=(PGRID,),
        in_specs=[_pspec2(), _pspec2(), _pspec(),
                  _wspec(), _wspec(), _wspec(), _wspec(), _bspec(), _bspec(),
                  _wspec(), _wspec(), _bspec(), _wspec(), _bspec(),
                  _wspec(), _wspec(), _bspec(), _wspec(), _bspec(),
                  _wspec(), _bspec()],
        out_specs=os,
        out_shape=osh,
    )(acc_p, xs_p, dinv_p, *BDs, bl, bh, Ml, Mh, bm1, M2, bm2,
      Gl, Gh, bg1, G2, bg2, SH, one6)


def _z_body(pool_ref, eps_ref, mu_ref, sg_ref, zp_ref):
    p = pool_ref[0] + pool_ref[1]
    denom = jnp.maximum(p[:, 6:7], 1.0)
    mu = p[:, 0:3] / denom
    sg = p[:, 3:6] / denom
    z = mu + eps_ref[...] * jnp.exp(0.5 * sg)
    mu_ref[...] = mu
    sg_ref[...] = sg
    zp_ref[...] = jnp.concatenate([z, jnp.zeros((G, 13), _f32)], axis=1)


def _z_kernel(pooled, eps):
    return pl.pallas_call(
        _z_body,
        in_specs=[pl.BlockSpec((2, G, 16), lambda: (0, 0, 0)),
                  pl.BlockSpec((G, 3), lambda: (0, 0))],
        out_specs=[pl.BlockSpec((G, 3), lambda: (0, 0)),
                   pl.BlockSpec((G, 3), lambda: (0, 0)),
                   pl.BlockSpec((G, 16), lambda: (0, 0))],
        out_shape=[jax.ShapeDtypeStruct((G, 3), _f32),
                   jax.ShapeDtypeStruct((G, 3), _f32),
                   jax.ShapeDtypeStruct((G, 16), _f32)],
    )(pooled, eps)


def _dec_head_body(zn_ref, dinv_ref, U1_ref, bu1_ref, U2l_ref, U2h_ref,
                   b2l_ref, b2h_ref, out_ref):
    dinv = dinv_ref[...]
    h1 = jnp.maximum(_dot(zn_ref[...], U1_ref[...]) + bu1_ref[...], 0.0)
    out_ref[0] = dinv * jnp.maximum(_dot(h1, U2l_ref[...]) + b2l_ref[...],
                                    0.0)
    out_ref[1] = dinv * jnp.maximum(_dot(h1, U2h_ref[...]) + b2h_ref[...],
                                    0.0)


def _dec_head(zn_p, dinv_p, U1, bu1, U2l, U2h, b2l, b2h):
    os, osh = _pout(2)
    return pl.pallas_call(
        _dec_head_body,
        grid=(PGRID,),
        in_specs=[_pspec(), _pspec(), _wspec(), _bspec(),
                  _wspec(), _wspec(), _bspec(), _bspec()],
        out_specs=os,
        out_shape=osh,
    )(zn_p, dinv_p, U1, bu1, U2l, U2h, b2l, b2h)


def _dec3_body(acc_ref, xs_ref, dinv_ref, Wll_ref, Whl_ref, Wlh_ref, Whh_ref,
               bl_ref, bh_ref, W4l_ref, W4h_ref, out_ref):
    dinv = dinv_ref[...]
    h0, h1 = _hidden32(acc_ref, xs_ref, dinv, Wll_ref[...], Whl_ref[...],
                       Wlh_ref[...], Whh_ref[...], bl_ref[...], bh_ref[...])
    out_ref[...] = _dot(dinv * h0, W4l_ref[...]) + _dot(dinv * h1,
                                                        W4h_ref[...])


def _dec3(acc_p, xs_p, dinv_p, BDs, bl, bh, W4l, W4h):
    os, osh = _pout()
    return pl.pallas_call(
        _dec3_body,
        grid=(PGRID,),
        in_specs=[_pspec2(), _pspec2(), _pspec(),
                  _wspec(), _wspec(), _wspec(), _wspec(), _bspec(), _bspec(),
                  _wspec(), _wspec()],
        out_specs=os,
        out_shape=osh,
    )(acc_p, xs_p, dinv_p, *BDs, bl, bh, W4l, W4h)


def _final_body(acc_ref, ys_ref, dinv_ref, b_ref, out_ref):
    t = dinv_ref[...] * (acc_ref[0] + acc_ref[1] + ys_ref[...]) + b_ref[...]
    out_ref[...] = jnp.maximum(t, 0.0)


def _final(acc_p, ys_p, dinv_p, b4t):
    os, osh = _pout()
    return pl.pallas_call(
        _final_body,
        grid=(PGRID,),
        in_specs=[_pspec2(), _pspec(), _pspec(), _bspec()],
        out_specs=os,
        out_shape=osh,
    )(acc_p, ys_p, dinv_p, b4t)


def _edge_pass(xs, sidi, col_split):
    n_chunks = (IROWS // 16 if col_split else IROWS // 32) // CWIN

    @functools.partial(
        pl.kernel,
        compiler_params=pltpu.CompilerParams(use_tc_tiling_on_sc=False),
        out_type=jax.ShapeDtypeStruct((2, NPK, 16), _f32),
        mesh=_mesh,
        scratch_types=[
            pltpu.VMEM((2, 2, CWIN * 128), jnp.int32),
            pltpu.VMEM((2, CWIN * 128, 16), _f32),
            pltpu.VMEM((128, 16), _f32),
            pltpu.VMEM_SHARED((ACC_R, 16), _f32),
            pltpu.SemaphoreType.DMA,
            pltpu.SemaphoreType.DMA,
            pltpu.SemaphoreType.DMA,
            pltpu.SemaphoreType.DMA,
            pltpu.SemaphoreType.DMA,
        ],
    )
    def k(xs_hbm, sidi_hbm, out_hbm,
          idx_v, rows_v, zb_v, acc,
          sem_i0, sem_i1, sem_g, sem_s0, sem_s1):
        c = lax.axis_index("c")
        s = lax.axis_index("s")

        @pl.loop(0, 128)
        def _fill(i):
            zb_v[i, :] = jnp.zeros((16,), _f32)

        @pl.loop(0, 49)
        def _zero(i):
            pltpu.sync_copy(zb_v, acc.at[pl.ds(s * 6272 + i * 128, 128)])

        plsc.subcore_barrier()

        ew = CWIN * 128
        base_e = (s * 1568 if col_split else (s * 2 + c) * 784) * 128
        sem_i = (sem_i0, sem_i1)
        sem_s = (sem_s0, sem_s1)
        src_tbl = xs_hbm.at[c] if col_split else xs_hbm
        last = n_chunks - 1

        def stage_idx(i, b):
            e0 = base_e + jnp.minimum(i, last) * ew
            pltpu.async_copy(sidi_hbm.at[:, pl.ds(e0, ew)], idx_v.at[b],
                             sem_i[b])

        def wait_idx(i, b):
            e0 = base_e + jnp.minimum(i, last) * ew
            pltpu.make_async_copy(sidi_hbm.at[:, pl.ds(e0, ew)], idx_v.at[b],
                                  sem_i[b]).wait()

        def wait_scat(b):
            pltpu.make_async_copy(rows_v.at[b], acc.at[idx_v.at[b].at[1]],
                                  sem_s[b]).wait()

        def do_chunk(i, b, first):
            if not first:
                wait_scat(b)
            wait_idx(i, b)
            pltpu.async_copy(src_tbl.at[idx_v.at[b].at[0]], rows_v.at[b],
                             sem_g).wait()
            pltpu.async_copy(rows_v.at[b], acc.at[idx_v.at[b].at[1]],
                             sem_s[b], add=True)
            stage_idx(i + 2, b)

        stage_idx(0, 0)
        stage_idx(1, 1)
        do_chunk(0, 0, True)
        do_chunk(1, 1, True)

        @pl.loop(2, n_chunks)
        def _c(i):
            @pl.when(i % 2 == 0)
            def _even():
                do_chunk(i, 0, False)

            @pl.when(i % 2 == 1)
            def _odd():
                do_chunk(i, 1, False)

        wait_scat(0)
        wait_scat(1)
        wait_idx(n_chunks, 0)
        wait_idx(n_chunks + 1, 1)
        plsc.subcore_barrier()
        pltpu.sync_copy(acc.at[pl.ds(s * 6272, 6272)],
                        out_hbm.at[c].at[pl.ds(s * 6272, 6272)])

    return k(xs, sidi)


def _sc_deg(di_pad):
    n_chunks = (IROWS // 32) // 8

    @functools.partial(
        pl.kernel,
        compiler_params=pltpu.CompilerParams(use_tc_tiling_on_sc=False),
        out_type=[jax.ShapeDtypeStruct((ACC_R,), _f32),
                  jax.ShapeDtypeStruct((ACC_R,), _f32)],
        mesh=_mesh,
        scratch_types=[
            pltpu.VMEM((8, 128), jnp.int32),
            pltpu.VMEM((128,), _f32),
            pltpu.VMEM((784,), _f32),
            pltpu.VMEM_SHARED((ACC_R,), _f32),
            pltpu.SemaphoreType.DMA,
            pltpu.SemaphoreType.DMA,
        ],
    )
    def k(di_hbm, out0_hbm, out1_hbm, di_v, ones_v, zb_v, acc, sem_i, sem_s):
        c = lax.axis_index("c")
        s = lax.axis_index("s")

        @pl.loop(0, 8)
        def _fill1(i):
            ones_v[pl.ds(i * 16, 16)] = jnp.ones((16,), _f32)

        @pl.loop(0, 49)
        def _fill0(i):
            zb_v[pl.ds(i * 16, 16)] = jnp.zeros((16,), _f32)

        @pl.loop(0, 8)
        def _zero(i):
            pltpu.sync_copy(zb_v, acc.at[pl.ds(s * 6272 + i * 784, 784)])

        plsc.subcore_barrier()
        base_row = (s * 2 + c) * 784

        @pl.loop(0, n_chunks)
        def _chunk(i):
            row0 = base_row + i * 8
            pltpu.async_copy(di_hbm.at[pl.ds(row0, 8)], di_v, sem_i).wait()
            ss = [pltpu.async_copy(ones_v, acc.at[di_v.at[j]], sem_s,
                                   add=True) for j in range(8)]
            for t in ss:
                t.wait()

        plsc.subcore_barrier()

        @pl.when(c == 0)
        def _drain0():
            pltpu.sync_copy(acc.at[pl.ds(s * 6272, 6272)],
                            out0_hbm.at[pl.ds(s * 6272, 6272)])

        @pl.when(c == 1)
        def _drain1():
            pltpu.sync_copy(acc.at[pl.ds(s * 6272, 6272)],
                            out1_hbm.at[pl.ds(s * 6272, 6272)])

    return k(di_pad)


def _sc_pool(combined_pad, bi_rows):
    @functools.partial(
        pl.kernel,
        compiler_params=pltpu.CompilerParams(use_tc_tiling_on_sc=False),
        out_type=jax.ShapeDtypeStruct((2, G, 16), _f32),
        mesh=_mesh,
        scratch_types=[
            pltpu.VMEM((32, 128), jnp.int32),
            pltpu.VMEM((4096, 16), _f32),
            pltpu.VMEM((64, 16), _f32),
            pltpu.VMEM_SHARED((1024, 16), _f32),
            pltpu.SemaphoreType.DMA,
            pltpu.SemaphoreType.DMA,
        ],
    )
    def k(comb_hbm, bi_hbm, out_hbm, bi_v, rows_v, zb_v, acc, sem_i, sem_s):
        c = lax.axis_index("c")
        s = lax.axis_index("s")

        @pl.loop(0, 64)
        def _fill(i):
            zb_v[i, :] = jnp.zeros((16,), _f32)

        pltpu.sync_copy(zb_v, acc.at[pl.ds(s * 64, 64)])
        plsc.subcore_barrier()

        w = s * 2 + c
        ci = pltpu.async_copy(bi_hbm.at[pl.ds(w * 32, 32)], bi_v, sem_i)
        cr = pltpu.async_copy(comb_hbm.at[pl.ds(w * 4096, 4096)], rows_v,
                              sem_i)
        ci.wait()
        cr.wait()
        ss = [pltpu.async_copy(rows_v.at[pl.ds(j * 128, 128)],
                               acc.at[bi_v.at[j]], sem_s, add=True)
              for j in range(32)]
        for t in ss:
            t.wait()
        plsc.subcore_barrier()

        @pl.when(s == 0)
        def _drain():
            pltpu.sync_copy(acc.at[pl.ds(0, G)], out_hbm.at[c])

    return k(combined_pad, bi_rows)


def _sc_zn(zpad, bi_rows):
    @functools.partial(
        pl.kernel,
        compiler_params=pltpu.CompilerParams(use_tc_tiling_on_sc=False),
        out_type=jax.ShapeDtypeStruct((NP, 16), _f32),
        mesh=_mesh,
        scratch_types=[
            pltpu.VMEM((32, 128), jnp.int32),
            pltpu.VMEM((4096, 16), _f32),
            pltpu.SemaphoreType.DMA,
            pltpu.SemaphoreType.DMA,
        ],
    )
    def k(z_hbm, bi_hbm, out_hbm, bi_v, rows_v, sem_i, sem_g):
        c = lax.axis_index("c")
        s = lax.axis_index("s")
        w = s * 2 + c
        pltpu.async_copy(bi_hbm.at[pl.ds(w * 32, 32)], bi_v, sem_i).wait()
        gs = [pltpu.async_copy(z_hbm.at[bi_v.at[j]],
                               rows_v.at[pl.ds(j * 128, 128)], sem_g)
              for j in range(32)]
        for g in gs:
            g.wait()
        pltpu.sync_copy(rows_v, out_hbm.at[pl.ds(w * 4096, 4096)])

    return k(zpad, bi_rows)


def _k8(M):
    return jnp.kron(jnp.eye(8, dtype=_f32), M.astype(_f32))


def _t8(v):
    return jnp.tile(v.astype(_f32), 8).reshape(1, 128)


def _bd4(W):
    return (_k8(W[:16, :16]), _k8(W[16:, :16]),
            _k8(W[:16, 16:]), _k8(W[16:, 16:]))


def kernel(x, edge_index, batch_index,
           enc_W1, enc_b1, enc_W2, enc_b2, enc_W3, enc_b3, enc_W4, enc_b4,
           mu_W1, mu_b1, mu_W2, mu_b2,
           sg_W1, sg_b1, sg_W2, sg_b2,
           un_W1, un_b1, un_W2, un_b2,
           dec_W1, dec_b1, dec_W2, dec_b2, dec_W3, dec_b3, dec_W4, dec_b4):
    src = edge_index[0].astype(jnp.int32)
    dst = edge_index[1].astype(jnp.int32)

    W1p = jnp.zeros((16, 32), _f32).at[:5].set(enc_W1)
    BD1l, BD1h = _k8(W1p[:, :16]), _k8(W1p[:, 16:])
    b1l, b1h = _t8(enc_b1[:16]), _t8(enc_b1[16:])
    enc_BDs = [(_bd4(W), _t8(b[:16]), _t8(b[16:]))
               for W, b in ((enc_W2, enc_b2), (enc_W3, enc_b3))]
    BD4, b4l, b4h = _bd4(enc_W4), _t8(enc_b4[:16]), _t8(enc_b4[16:])
    Ml, Mh = _k8(mu_W1[:16, :]), _k8(mu_W1[16:, :])
    bm1 = _t8(mu_b1)
    M2 = _k8(jnp.zeros((16, 16), _f32).at[:, :3].set(mu_W2))
    bm2 = _t8(jnp.zeros((16,), _f32).at[:3].set(mu_b2))
    Gl, Gh = _k8(sg_W1[:16, :]), _k8(sg_W1[16:, :])
    bg1 = _t8(sg_b1)
    G2 = _k8(jnp.zeros((16, 16), _f32).at[:, :3].set(sg_W2))
    bg2 = _t8(jnp.zeros((16,), _f32).at[:3].set(sg_b2))
    SH = _k8(jnp.zeros((16, 16), _f32).at[jnp.arange(3),
                                          jnp.arange(3) + 3].set(1.0))
    one6 = _t8(jnp.zeros((16,), _f32).at[6].set(1.0))
    U1 = _k8(jnp.zeros((16, 16), _f32).at[:3].set(un_W1))
    bu1 = _t8(un_b1)
    U2l, U2h = _k8(un_W2[:, :16]), _k8(un_W2[:, 16:])
    bu2l, bu2h = _t8(un_b2[:16]), _t8(un_b2[16:])
    dec_BDs = [(_bd4(W), _t8(b[:16]), _t8(b[16:]))
               for W, b in ((dec_W1, dec_b1), (dec_W2, dec_b2))]
    BDd3, bd3l, bd3h = _bd4(dec_W3), _t8(dec_b3[:16]), _t8(dec_b3[16:])
    decW4p = jnp.zeros((32, 16), _f32).at[:, :5].set(dec_W4)
    W4l, W4h = _k8(decW4p[:16, :]), _k8(decW4p[16:, :])
    b4t = _t8(jnp.zeros((16,), _f32).at[:5].set(dec_b4))
    eps = jax.random.normal(jax.random.key(42), (G, 3), dtype=_f32)

    npad = E_PAD - E
    si_pad = jnp.concatenate([src, jnp.zeros((npad,), jnp.int32)])
    di_pad = jnp.concatenate([dst, jnp.full((npad,), DUMMY, jnp.int32)])
    sidi = jnp.stack([si_pad, di_pad])
    bi_pad = jnp.concatenate(
        [batch_index.astype(jnp.int32),
         jnp.full((NP - N,), GDUM, jnp.int32)]).reshape(1024, 128)

    deg0, deg1 = _sc_deg(di_pad.reshape(IROWS, 128))
    deg0_p = jnp.repeat(deg0, 16).reshape(PR, 128)
    deg1_p = jnp.repeat(deg1, 16).reshape(PR, 128)
    xpad_p = jnp.zeros((NPK, 16), _f32).at[:N, :5].set(x).reshape(PR, 128)
    dinv_p, xs0_p = _prep(deg0_p, deg1_p, xpad_p)

    acc = _edge_pass(xs0_p.reshape(NPK, 16), sidi, col_split=False)
    xs_p = _layer16(acc.reshape(2, PR, 128), xs0_p, dinv_p,
                    BD1l, BD1h, b1l, b1h)
    for BDs, bl, bh in enc_BDs:
        acc = _edge_pass(xs_p.reshape(2, NPK, 16), sidi, col_split=True)
        xs_p = _layer32(acc.reshape(2, PR, 128), xs_p, dinv_p, BDs, bl, bh)
    acc = _edge_pass(xs_p.reshape(2, NPK, 16), sidi, col_split=True)
    combined_p = _enc4_head(acc.reshape(2, PR, 128), xs_p, dinv_p,
                            BD4, b4l, b4h, Ml, Mh, bm1, M2, bm2,
                            Gl, Gh, bg1, G2, bg2, SH, one6)

    combined_pad = jnp.concatenate(
        [combined_p.reshape(NPK, 16), jnp.zeros((NP - NPK, 16), _f32)])
    pooled = _sc_pool(combined_pad, bi_pad)
    mu, sigma, zpad = _z_kernel(pooled, eps)

    zpad2 = jnp.concatenate([zpad, jnp.zeros((8, 16), _f32)])
    zn_p = _sc_zn(zpad2, bi_pad)[:NPK].reshape(PR, 128)
    xs_p = _dec_head(zn_p, dinv_p, U1, bu1, U2l, U2h, bu2l, bu2h)
    for BDs, bl, bh in dec_BDs:
        acc = _edge_pass(xs_p.reshape(2, NPK, 16), sidi, col_split=True)
        xs_p = _layer32(acc.reshape(2, PR, 128), xs_p, dinv_p, BDs, bl, bh)
    acc = _edge_pass(xs_p.reshape(2, NPK, 16), sidi, col_split=True)
    ys_p = _dec3(acc.reshape(2, PR, 128), xs_p, dinv_p, BDd3, bd3l, bd3h,
                 W4l, W4h)
    acc = _edge_pass(ys_p.reshape(NPK, 16), sidi, col_split=False)
    h2_p = _final(acc.reshape(2, PR, 128), ys_p, dinv_p, b4t)
    h2 = h2_p.reshape(NPK, 16)[:N, :5]
    return (h2, mu, sigma)
